# baseline (speedup 1.0000x reference)
import functools

import jax
import jax.numpy as jnp
from jax import lax
from jax.experimental import pallas as pl
from jax.experimental.pallas import tpu as pltpu
from jax.experimental.pallas import tpu_sc as plsc

B = 16384
NC = 2
NS = 16
NW = NC * NS
BPW = B // NW

DU = 32
DI = 32
DC = 16

_mesh = plsc.VectorSubcoreMesh(core_axis_name="c", subcore_axis_name="s")


@functools.partial(
    pl.kernel,
    mesh=_mesh,
    compiler_params=pltpu.CompilerParams(use_tc_tiling_on_sc=False),
    out_type=(
        jax.ShapeDtypeStruct((B, DU), jnp.float32),
        jax.ShapeDtypeStruct((B, DI), jnp.float32),
        jax.ShapeDtypeStruct((B, DC), jnp.float32),
    ),
    scratch_types=[
        pltpu.VMEM((BPW,), jnp.int32),
        pltpu.VMEM((BPW,), jnp.int32),
        pltpu.VMEM((BPW,), jnp.int32),
        pltpu.VMEM((BPW, DU), jnp.float32),
        pltpu.VMEM((BPW, DI), jnp.float32),
        pltpu.VMEM((BPW, DC), jnp.float32),
        pltpu.SemaphoreType.DMA,
        pltpu.SemaphoreType.DMA,
        pltpu.SemaphoreType.DMA,
    ],
)
def _sc_gather(uid_hbm, iid_hbm, cid_hbm, emb_u_hbm, emb_i_hbm, emb_c_hbm,
               out_u, out_i, out_c,
               idx_u, idx_i, idx_c, rows_u, rows_i, rows_c,
               sem_u, sem_i, sem_c):
    wid = lax.axis_index("s") * NC + lax.axis_index("c")
    base = wid * BPW
    pltpu.sync_copy(uid_hbm.at[pl.ds(base, BPW)], idx_u)
    pltpu.sync_copy(iid_hbm.at[pl.ds(base, BPW)], idx_i)
    pltpu.sync_copy(cid_hbm.at[pl.ds(base, BPW)], idx_c)
    cu = pltpu.async_copy(emb_u_hbm.at[idx_u], rows_u, sem_u)
    ci = pltpu.async_copy(emb_i_hbm.at[idx_i], rows_i, sem_i)
    cc = pltpu.async_copy(emb_c_hbm.at[idx_c], rows_c, sem_c)
    cu.wait()
    pltpu.sync_copy(rows_u, out_u.at[pl.ds(base, BPW)])
    ci.wait()
    pltpu.sync_copy(rows_i, out_i.at[pl.ds(base, BPW)])
    cc.wait()
    pltpu.sync_copy(rows_c, out_c.at[pl.ds(base, BPW)])


BLK = 2048


def _mlp_body(X_ref, eu_ref, ei_ref, ec_ref, tags_ref,
              WtT_ref, bt_ref, W1x_ref, W1u_ref, W1i_ref, W1c_ref, W1t_ref,
              b1_ref, W2T_ref, b2_ref, W3T_ref, b3_ref, out_ref):
    f32 = jnp.float32
    At = jnp.dot(WtT_ref[...], W1t_ref[...], preferred_element_type=f32)
    bias1 = b1_ref[...] + jnp.dot(bt_ref[...], W1t_ref[...],
                                  preferred_element_type=f32)
    h = jnp.dot(X_ref[...], W1x_ref[...], preferred_element_type=f32)
    h = h + jnp.dot(eu_ref[...], W1u_ref[...], preferred_element_type=f32)
    h = h + jnp.dot(ei_ref[...], W1i_ref[...], preferred_element_type=f32)
    h = h + jnp.dot(ec_ref[...], W1c_ref[...], preferred_element_type=f32)
    h = h + jnp.dot(tags_ref[...], At, preferred_element_type=f32)
    h = jnp.maximum(h + bias1, 0.0)
    h2 = jnp.maximum(
        jnp.dot(h, W2T_ref[...], preferred_element_type=f32) + b2_ref[...], 0.0)
    out_ref[...] = (jnp.dot(h2, W3T_ref[...], preferred_element_type=f32)
                    + b3_ref[...])


def _row_spec(width):
    return pl.BlockSpec((BLK, width), lambda i: (i, 0))


def _full_spec(r, c):
    return pl.BlockSpec((r, c), lambda i: (0, 0))


def kernel(X, user_id, item_id, category, tags, emb_user, emb_item, emb_cat,
           W_tags, b_tags, W1, b1, W2, b2, W3, b3):
    uid = user_id.astype(jnp.int32)
    iid = item_id.astype(jnp.int32)
    cid = category.astype(jnp.int32)

    e_user, e_item, e_cat = _sc_gather(uid, iid, cid, emb_user, emb_item,
                                       emb_cat)

    W1T = W1.T
    W1x = W1T[0:13]
    W1u = W1T[13:45]
    W1i = W1T[45:77]
    W1c = W1T[77:93]
    W1t = W1T[93:157]

    out = pl.pallas_call(
        _mlp_body,
        grid=(B // BLK,),
        in_specs=[
            _row_spec(13), _row_spec(DU), _row_spec(DI), _row_spec(DC),
            _row_spec(64),
            _full_spec(64, 64),
            _full_spec(1, 64),
            _full_spec(13, 64), _full_spec(32, 64), _full_spec(32, 64),
            _full_spec(16, 64), _full_spec(64, 64),
            _full_spec(1, 64),
            _full_spec(64, 16),
            _full_spec(1, 16),
            _full_spec(16, 1),
            _full_spec(1, 1),
        ],
        out_specs=_row_spec(1),
        out_shape=jax.ShapeDtypeStruct((B, 1), jnp.float32),
    )(X, e_user, e_item, e_cat, tags,
      W_tags.T, b_tags.reshape(1, 64),
      W1x, W1u, W1i, W1c, W1t,
      b1.reshape(1, 64), W2.T, b2.reshape(1, 16), W3.T, b3.reshape(1, 1))
    return out[:, 0]
